# Initial kernel scaffold; baseline (speedup 1.0000x reference)
import jax
import jax.numpy as jnp
from jax.experimental import pallas as pl


def kernel(x, edge_index, batch, params):
    raise NotImplementedError("write your pallas kernel here")


# speedup vs baseline: 6.4515x; 6.4515x over previous
import jax
import jax.numpy as jnp
from jax.experimental import pallas as pl
from jax.experimental.pallas import tpu as pltpu

N = 10000
E = 320000
D_IN = 128
HID = 64
HEADS = 32
KA = 16
G = 64
EPS = 0.2
ITERS = 30
NC = 2

BN = 400
NB = N // BN
BNC = 400
NBC = N // BNC
BM = 2000
NBM = N // BM


def _gin_mlp_kernel(h_ref, agg_ref, deg_ref, w1_ref, b1_ref, w2_ref, b2_ref, o_ref):
    z = h_ref[...] + agg_ref[...] / deg_ref[...]
    z1 = jnp.maximum(
        jnp.dot(z, w1_ref[...], preferred_element_type=jnp.float32) + b1_ref[...], 0.0)
    z2 = jnp.dot(z1, w2_ref[...], preferred_element_type=jnp.float32) + b2_ref[...]
    o_ref[...] = jnp.maximum(z2, 0.0)


def _gin_mlp(h, agg, deg, W1, b1, W2, b2):
    di = h.shape[1]
    do = W2.shape[1]
    return pl.pallas_call(
        _gin_mlp_kernel,
        grid=(NBM,),
        in_specs=[
            pl.BlockSpec((BM, di), lambda i: (i, 0)),
            pl.BlockSpec((BM, di), lambda i: (i, 0)),
            pl.BlockSpec((BM, 1), lambda i: (i, 0)),
            pl.BlockSpec((di, HID), lambda i: (0, 0)),
            pl.BlockSpec((1, HID), lambda i: (0, 0)),
            pl.BlockSpec((HID, do), lambda i: (0, 0)),
            pl.BlockSpec((1, do), lambda i: (0, 0)),
        ],
        out_specs=pl.BlockSpec((BM, do), lambda i: (i, 0)),
        out_shape=jax.ShapeDtypeStruct((N, do), jnp.float32),
    )(h, agg, deg, W1, b1, W2, b2)


def _kmat_kernel(h_ref, wh_ref, bh_ref, cbt_ref, ok_ref, op_ref):
    hb = h_ref[...]
    cbt = cbt_ref[...]
    c2 = jnp.sum(cbt * cbt, axis=0)[None, :]
    for hd in range(HEADS):
        d = jnp.dot(hb, wh_ref[hd], preferred_element_type=jnp.float32) + bh_ref[hd:hd + 1, :]
        op_ref[:, hd, :] = d
        p2 = jnp.sum(d * d, axis=1, keepdims=True)
        cr = jnp.dot(d, cbt, preferred_element_type=jnp.float32)
        C = p2 + c2 - 2.0 * cr
        C = C - jnp.min(C, axis=1, keepdims=True)
        ok_ref[:, hd, :] = jnp.exp(-C / EPS)


def _sinkhorn_kernel(k_ref, b_ref, a_ref, u_ref, v_ref, v_s, s_s):
    it = pl.program_id(0)
    nb = pl.program_id(1)

    @pl.when(jnp.logical_and(it == 0, nb == 0))
    def _():
        v_s[...] = jnp.ones((G, KA), jnp.float32)

    @pl.when(nb == 0)
    def _():
        s_s[...] = jnp.zeros((G, KA), jnp.float32)

    Bb = b_ref[...]
    v = v_s[...]
    Vn = jnp.dot(Bb, v, preferred_element_type=jnp.float32)
    K3 = k_ref[...]
    Kv = jnp.sum(K3 * Vn[:, None, :], axis=2)
    U = a_ref[...] / (Kv + 1e-9)
    u_ref[...] = U
    S = jnp.sum(K3 * U[:, :, None], axis=1)
    s_s[...] += jax.lax.dot_general(
        Bb, S, (((0,), (0,)), ((), ())), preferred_element_type=jnp.float32)

    @pl.when(nb == NB - 1)
    def _():
        nv = (1.0 / KA) / (s_s[...] + 1e-9)
        v_s[...] = nv
        v_ref[...] = nv


def _bary_kernel(k_ref, u_ref, vv_ref, b_ref, p_ref, w_ref, bc_ref):
    nb = pl.program_id(0)

    @pl.when(nb == 0)
    def _():
        w_ref[...] = jnp.zeros_like(w_ref)
        bc_ref[...] = jnp.zeros_like(bc_ref)

    Bb = b_ref[...]
    v = vv_ref[...]
    Vn = jnp.dot(Bb, v, preferred_element_type=jnp.float32)
    K3 = k_ref[...]
    U = u_ref[...]
    T = K3 * U[:, :, None] * Vn[:, None, :]
    Ts = jnp.sum(T, axis=1)
    w_ref[...] += jax.lax.dot_general(
        Bb, Ts, (((0,), (0,)), ((), ())), preferred_element_type=jnp.float32)
    Pts = p_ref[...]
    for k in range(KA):
        Tk = jax.lax.slice(T, (0, 0, k), (BNC, HEADS, k + 1))
        Ck = jnp.sum(Tk * Pts, axis=1)
        bc_ref[k] += jax.lax.dot_general(
            Bb, Ck, (((0,), (0,)), ((), ())), preferred_element_type=jnp.float32)


def _head_kernel(bc_ref, w_ref, r_ref,
                 w0_ref, b0_ref, w1_ref, b1_ref, w2_ref, b2_ref, w3_ref, b3_ref,
                 o_ref):
    w = w_ref[...]
    bcs = []
    scs = []
    for k in range(KA):
        bck = bc_ref[k] / (w[:, k:k + 1] + 1e-9)
        bcs.append(bck)
        scs.append(jnp.dot(bck, r_ref[...], preferred_element_type=jnp.float32))
    sc = jnp.concatenate(scs, axis=1)
    sc = sc - jnp.max(sc, axis=1, keepdims=True)
    al = jnp.exp(sc)
    al = al / jnp.sum(al, axis=1, keepdims=True)
    g = jnp.zeros((G, HID), jnp.float32)
    for k in range(KA):
        g = g + al[:, k:k + 1] * bcs[k]
    g = jnp.maximum(
        jnp.dot(g, w0_ref[...], preferred_element_type=jnp.float32) + b0_ref[...], 0.0)
    g = jnp.maximum(
        jnp.dot(g, w1_ref[...], preferred_element_type=jnp.float32) + b1_ref[...], 0.0)
    g = jnp.maximum(
        jnp.dot(g, w2_ref[...], preferred_element_type=jnp.float32) + b2_ref[...], 0.0)
    o_ref[...] = jnp.dot(g, w3_ref[...], preferred_element_type=jnp.float32) + b3_ref[...]


def kernel(x, edge_index, batch, params):
    src = edge_index[0]
    dst = edge_index[1]
    ones_e = jnp.ones((E,), jnp.float32)
    deg = jnp.maximum(jax.ops.segment_sum(ones_e, dst, num_segments=N), 1.0)[:, None]

    h = x
    for lp in params['gin']:
        agg = jax.ops.segment_sum(h[src], dst, num_segments=N)
        h = _gin_mlp(h, agg, deg, lp['W1'], lp['b1'][None, :], lp['W2'], lp['b2'][None, :])

    cbT = params['codebook'].T

    Km, pts = pl.pallas_call(
        _kmat_kernel,
        grid=(NBC,),
        in_specs=[
            pl.BlockSpec((BNC, HID), lambda i: (i, 0)),
            pl.BlockSpec((HEADS, HID, HID), lambda i: (0, 0, 0)),
            pl.BlockSpec((HEADS, HID), lambda i: (0, 0)),
            pl.BlockSpec((HID, KA), lambda i: (0, 0)),
        ],
        out_specs=[
            pl.BlockSpec((BNC, HEADS, KA), lambda i: (i, 0, 0)),
            pl.BlockSpec((BNC, HEADS, HID), lambda i: (i, 0, 0)),
        ],
        out_shape=[
            jax.ShapeDtypeStruct((N, HEADS, KA), jnp.float32),
            jax.ShapeDtypeStruct((N, HEADS, HID), jnp.float32),
        ],
    )(h, params['Wh'], params['bh'], cbT)

    Bmat = jax.nn.one_hot(batch, G, dtype=jnp.float32)
    ncnt = jnp.sum(Bmat, axis=0)
    Ainv = 1.0 / jnp.maximum(ncnt * HEADS, 1.0)
    Anode = Ainv[batch][:, None]

    u, v = pl.pallas_call(
        _sinkhorn_kernel,
        grid=(ITERS, NB),
        in_specs=[
            pl.BlockSpec((BN, HEADS, KA), lambda it, nb: (nb, 0, 0)),
            pl.BlockSpec((BN, G), lambda it, nb: (nb, 0)),
            pl.BlockSpec((BN, 1), lambda it, nb: (nb, 0)),
        ],
        out_specs=[
            pl.BlockSpec((BN, HEADS), lambda it, nb: (nb, 0)),
            pl.BlockSpec((G, KA), lambda it, nb: (0, 0)),
        ],
        out_shape=[
            jax.ShapeDtypeStruct((N, HEADS), jnp.float32),
            jax.ShapeDtypeStruct((G, KA), jnp.float32),
        ],
        scratch_shapes=[
            pltpu.VMEM((G, KA), jnp.float32),
            pltpu.VMEM((G, KA), jnp.float32),
        ],
    )(Km, Bmat, Anode)

    w, bc = pl.pallas_call(
        _bary_kernel,
        grid=(NBC,),
        in_specs=[
            pl.BlockSpec((BNC, HEADS, KA), lambda i: (i, 0, 0)),
            pl.BlockSpec((BNC, HEADS), lambda i: (i, 0)),
            pl.BlockSpec((G, KA), lambda i: (0, 0)),
            pl.BlockSpec((BNC, G), lambda i: (i, 0)),
            pl.BlockSpec((BNC, HEADS, HID), lambda i: (i, 0, 0)),
        ],
        out_specs=[
            pl.BlockSpec((G, KA), lambda i: (0, 0)),
            pl.BlockSpec((KA, G, HID), lambda i: (0, 0, 0)),
        ],
        out_shape=[
            jax.ShapeDtypeStruct((G, KA), jnp.float32),
            jax.ShapeDtypeStruct((KA, G, HID), jnp.float32),
        ],
    )(Km, u, v, Bmat, pts)

    cls = params['cls']
    out = pl.pallas_call(
        _head_kernel,
        out_shape=jax.ShapeDtypeStruct((G, NC), jnp.float32),
    )(bc, w, params['r'][:, None],
      cls[0]['W'], cls[0]['b'][None, :],
      cls[1]['W'], cls[1]['b'][None, :],
      cls[2]['W'], cls[2]['b'][None, :],
      cls[3]['W'], cls[3]['b'][None, :])
    return out


# speedup vs baseline: 6.5657x; 1.0177x over previous
import jax
import jax.numpy as jnp
from jax.experimental import pallas as pl
from jax.experimental.pallas import tpu as pltpu

N = 10000
E = 320000
D_IN = 128
HID = 64
HEADS = 32
KA = 16
G = 64
EPS = 0.2
ITERS = 30
NC = 2

BN = 1000
NB = N // BN
BNC = 400
NBC = N // BNC
BM = 2000
NBM = N // BM


def _gin_mlp_kernel(h_ref, agg_ref, deg_ref, w1_ref, b1_ref, w2_ref, b2_ref, o_ref):
    z = h_ref[...] + agg_ref[...] / deg_ref[...]
    z1 = jnp.maximum(
        jnp.dot(z, w1_ref[...], preferred_element_type=jnp.float32) + b1_ref[...], 0.0)
    z2 = jnp.dot(z1, w2_ref[...], preferred_element_type=jnp.float32) + b2_ref[...]
    o_ref[...] = jnp.maximum(z2, 0.0)


def _gin_mlp(h, agg, deg, W1, b1, W2, b2):
    di = h.shape[1]
    do = W2.shape[1]
    return pl.pallas_call(
        _gin_mlp_kernel,
        grid=(NBM,),
        in_specs=[
            pl.BlockSpec((BM, di), lambda i: (i, 0)),
            pl.BlockSpec((BM, di), lambda i: (i, 0)),
            pl.BlockSpec((BM, 1), lambda i: (i, 0)),
            pl.BlockSpec((di, HID), lambda i: (0, 0)),
            pl.BlockSpec((1, HID), lambda i: (0, 0)),
            pl.BlockSpec((HID, do), lambda i: (0, 0)),
            pl.BlockSpec((1, do), lambda i: (0, 0)),
        ],
        out_specs=pl.BlockSpec((BM, do), lambda i: (i, 0)),
        out_shape=jax.ShapeDtypeStruct((N, do), jnp.float32),
    )(h, agg, deg, W1, b1, W2, b2)


def _kmat_kernel(h_ref, wh_ref, bh_ref, cbt_ref, ok_ref, op_ref):
    hb = h_ref[...]
    cbt = cbt_ref[...]
    c2 = jnp.sum(cbt * cbt, axis=0)[None, :]
    for hd in range(HEADS):
        d = jnp.dot(hb, wh_ref[hd], preferred_element_type=jnp.float32) + bh_ref[hd:hd + 1, :]
        op_ref[:, hd, :] = d
        p2 = jnp.sum(d * d, axis=1, keepdims=True)
        cr = jnp.dot(d, cbt, preferred_element_type=jnp.float32)
        C = p2 + c2 - 2.0 * cr
        C = C - jnp.min(C, axis=1, keepdims=True)
        ok_ref[:, hd, :] = jnp.exp(-C / EPS)


def _sinkhorn_kernel(k_ref, b_ref, a_ref, u_ref, v_ref, v_s, s_s):
    it = pl.program_id(0)
    nb = pl.program_id(1)

    @pl.when(jnp.logical_and(it == 0, nb == 0))
    def _():
        v_s[...] = jnp.ones((G, KA), jnp.float32)

    @pl.when(nb == 0)
    def _():
        s_s[...] = jnp.zeros((G, KA), jnp.float32)

    Bb = b_ref[...]
    v = v_s[...]
    Vn = jnp.dot(Bb, v, preferred_element_type=jnp.float32)
    K3 = k_ref[...]
    Kv = jnp.sum(K3 * Vn[:, None, :], axis=2)
    U = a_ref[...] / (Kv + 1e-9)
    u_ref[...] = U
    S = jnp.sum(K3 * U[:, :, None], axis=1)
    s_s[...] += jax.lax.dot_general(
        Bb, S, (((0,), (0,)), ((), ())), preferred_element_type=jnp.float32)

    @pl.when(nb == NB - 1)
    def _():
        nv = (1.0 / KA) / (s_s[...] + 1e-9)
        v_s[...] = nv
        v_ref[...] = nv


def _bary_kernel(k_ref, u_ref, vv_ref, b_ref, p_ref, w_ref, bc_ref):
    nb = pl.program_id(0)

    @pl.when(nb == 0)
    def _():
        w_ref[...] = jnp.zeros_like(w_ref)
        bc_ref[...] = jnp.zeros_like(bc_ref)

    Bb = b_ref[...]
    v = vv_ref[...]
    Vn = jnp.dot(Bb, v, preferred_element_type=jnp.float32)
    K3 = k_ref[...]
    U = u_ref[...]
    T = K3 * U[:, :, None] * Vn[:, None, :]
    Ts = jnp.sum(T, axis=1)
    w_ref[...] += jax.lax.dot_general(
        Bb, Ts, (((0,), (0,)), ((), ())), preferred_element_type=jnp.float32)
    Pts = p_ref[...]
    for k in range(KA):
        Tk = jax.lax.slice(T, (0, 0, k), (BNC, HEADS, k + 1))
        Ck = jnp.sum(Tk * Pts, axis=1)
        bc_ref[k] += jax.lax.dot_general(
            Bb, Ck, (((0,), (0,)), ((), ())), preferred_element_type=jnp.float32)


def _head_kernel(bc_ref, w_ref, r_ref,
                 w0_ref, b0_ref, w1_ref, b1_ref, w2_ref, b2_ref, w3_ref, b3_ref,
                 o_ref):
    w = w_ref[...]
    bcs = []
    scs = []
    for k in range(KA):
        bck = bc_ref[k] / (w[:, k:k + 1] + 1e-9)
        bcs.append(bck)
        scs.append(jnp.dot(bck, r_ref[...], preferred_element_type=jnp.float32))
    sc = jnp.concatenate(scs, axis=1)
    sc = sc - jnp.max(sc, axis=1, keepdims=True)
    al = jnp.exp(sc)
    al = al / jnp.sum(al, axis=1, keepdims=True)
    g = jnp.zeros((G, HID), jnp.float32)
    for k in range(KA):
        g = g + al[:, k:k + 1] * bcs[k]
    g = jnp.maximum(
        jnp.dot(g, w0_ref[...], preferred_element_type=jnp.float32) + b0_ref[...], 0.0)
    g = jnp.maximum(
        jnp.dot(g, w1_ref[...], preferred_element_type=jnp.float32) + b1_ref[...], 0.0)
    g = jnp.maximum(
        jnp.dot(g, w2_ref[...], preferred_element_type=jnp.float32) + b2_ref[...], 0.0)
    o_ref[...] = jnp.dot(g, w3_ref[...], preferred_element_type=jnp.float32) + b3_ref[...]


def kernel(x, edge_index, batch, params):
    src = edge_index[0]
    dst = edge_index[1]
    ones_e = jnp.ones((E,), jnp.float32)
    deg = jnp.maximum(jax.ops.segment_sum(ones_e, dst, num_segments=N), 1.0)[:, None]

    h = x
    for lp in params['gin']:
        agg = jax.ops.segment_sum(h[src], dst, num_segments=N)
        h = _gin_mlp(h, agg, deg, lp['W1'], lp['b1'][None, :], lp['W2'], lp['b2'][None, :])

    cbT = params['codebook'].T

    Km, pts = pl.pallas_call(
        _kmat_kernel,
        grid=(NBC,),
        in_specs=[
            pl.BlockSpec((BNC, HID), lambda i: (i, 0)),
            pl.BlockSpec((HEADS, HID, HID), lambda i: (0, 0, 0)),
            pl.BlockSpec((HEADS, HID), lambda i: (0, 0)),
            pl.BlockSpec((HID, KA), lambda i: (0, 0)),
        ],
        out_specs=[
            pl.BlockSpec((BNC, HEADS, KA), lambda i: (i, 0, 0)),
            pl.BlockSpec((BNC, HEADS, HID), lambda i: (i, 0, 0)),
        ],
        out_shape=[
            jax.ShapeDtypeStruct((N, HEADS, KA), jnp.float32),
            jax.ShapeDtypeStruct((N, HEADS, HID), jnp.float32),
        ],
    )(h, params['Wh'], params['bh'], cbT)

    Bmat = jax.nn.one_hot(batch, G, dtype=jnp.float32)
    ncnt = jnp.sum(Bmat, axis=0)
    Ainv = 1.0 / jnp.maximum(ncnt * HEADS, 1.0)
    Anode = Ainv[batch][:, None]

    u, v = pl.pallas_call(
        _sinkhorn_kernel,
        grid=(ITERS, NB),
        in_specs=[
            pl.BlockSpec((BN, HEADS, KA), lambda it, nb: (nb, 0, 0)),
            pl.BlockSpec((BN, G), lambda it, nb: (nb, 0)),
            pl.BlockSpec((BN, 1), lambda it, nb: (nb, 0)),
        ],
        out_specs=[
            pl.BlockSpec((BN, HEADS), lambda it, nb: (nb, 0)),
            pl.BlockSpec((G, KA), lambda it, nb: (0, 0)),
        ],
        out_shape=[
            jax.ShapeDtypeStruct((N, HEADS), jnp.float32),
            jax.ShapeDtypeStruct((G, KA), jnp.float32),
        ],
        scratch_shapes=[
            pltpu.VMEM((G, KA), jnp.float32),
            pltpu.VMEM((G, KA), jnp.float32),
        ],
    )(Km, Bmat, Anode)

    w, bc = pl.pallas_call(
        _bary_kernel,
        grid=(NBC,),
        in_specs=[
            pl.BlockSpec((BNC, HEADS, KA), lambda i: (i, 0, 0)),
            pl.BlockSpec((BNC, HEADS), lambda i: (i, 0)),
            pl.BlockSpec((G, KA), lambda i: (0, 0)),
            pl.BlockSpec((BNC, G), lambda i: (i, 0)),
            pl.BlockSpec((BNC, HEADS, HID), lambda i: (i, 0, 0)),
        ],
        out_specs=[
            pl.BlockSpec((G, KA), lambda i: (0, 0)),
            pl.BlockSpec((KA, G, HID), lambda i: (0, 0, 0)),
        ],
        out_shape=[
            jax.ShapeDtypeStruct((G, KA), jnp.float32),
            jax.ShapeDtypeStruct((KA, G, HID), jnp.float32),
        ],
    )(Km, u, v, Bmat, pts)

    cls = params['cls']
    out = pl.pallas_call(
        _head_kernel,
        out_shape=jax.ShapeDtypeStruct((G, NC), jnp.float32),
    )(bc, w, params['r'][:, None],
      cls[0]['W'], cls[0]['b'][None, :],
      cls[1]['W'], cls[1]['b'][None, :],
      cls[2]['W'], cls[2]['b'][None, :],
      cls[3]['W'], cls[3]['b'][None, :])
    return out
